# speedup vs baseline: 4.4314x; 4.4314x over previous
import jax
import jax.numpy as jnp
from jax.experimental import pallas as pl

_N = 1000
_D = 32
_E = 100


def _body(x_ref, ps_ref, pd_ref, ns_ref, nd_ref,
          w1pl_ref, w1pr_ref, b1p_ref,
          w1nl_ref, w1nr_ref, b1n_ref,
          w2pla_ref, w2plb_ref, w2pr_ref, b2p_ref,
          w2nla_ref, w2nlb_ref, w2nr_ref, b2n_ref,
          out_ref):
    f32 = jnp.float32

    def dot(a, b):
        return jax.lax.dot(a, b, precision=jax.lax.Precision.HIGHEST,
                           preferred_element_type=f32)

    x = x_ref[...]

    iota_en = jax.lax.broadcasted_iota(jnp.int32, (_E, _N), 1)
    iota_ne = jax.lax.broadcasted_iota(jnp.int32, (_N, _E), 0)

    def edge_mats(s_ref, d_ref):
        src = s_ref[...]
        dst = d_ref[...]
        gat = (src == iota_en).astype(f32)
        sca = (iota_ne == dst).astype(f32)
        inv = 1.0 / jnp.maximum(jnp.sum(sca, axis=1, keepdims=True), 1.0)
        return gat, sca, inv

    gp, sp, ip = edge_mats(ps_ref, pd_ref)
    gn, sn, im = edge_mats(ns_ref, nd_ref)

    aggp = dot(sp, dot(gp, x)) * ip
    aggn = dot(sn, dot(gn, x)) * im

    x1 = jnp.maximum(dot(aggp, w1pl_ref[...]) + dot(x, w1pr_ref[...]) + b1p_ref[...], 0.0)
    x2 = jnp.maximum(dot(aggn, w1nl_ref[...]) + dot(x, w1nr_ref[...]) + b1n_ref[...], 0.0)

    z = jnp.concatenate([x1, x2], axis=-1)
    bp = dot(sp, dot(gp, z)) * ip
    bn = dot(sn, dot(gn, z)) * im

    op = (dot(bp[:, :_D], w2pla_ref[...]) + dot(bn[:, _D:], w2plb_ref[...])
          + dot(x1, w2pr_ref[...]) + b2p_ref[...])
    on = (dot(bp[:, _D:], w2nla_ref[...]) + dot(bn[:, :_D], w2nlb_ref[...])
          + dot(x2, w2nr_ref[...]) + b2n_ref[...])

    out_ref[...] = jnp.maximum(jnp.concatenate([op, on], axis=-1), 0.0)


def kernel(x, pos_edge_index, neg_edge_index,
           w1_pos_l, w1_pos_r, b1_pos_r,
           w1_neg_l, w1_neg_r, b1_neg_r,
           w2_pos_l, w2_pos_r, b2_pos_r,
           w2_neg_l, w2_neg_r, b2_neg_r):
    ps = pos_edge_index[0].reshape(_E, 1)
    pd = pos_edge_index[1].reshape(1, _E)
    ns = neg_edge_index[0].reshape(_E, 1)
    nd = neg_edge_index[1].reshape(1, _E)
    args = (x, ps, pd, ns, nd,
            w1_pos_l.T, w1_pos_r.T, b1_pos_r.reshape(1, _D),
            w1_neg_l.T, w1_neg_r.T, b1_neg_r.reshape(1, _D),
            w2_pos_l.T[:_D], w2_pos_l.T[_D:], w2_pos_r.T, b2_pos_r.reshape(1, _D),
            w2_neg_l.T[:_D], w2_neg_l.T[_D:], w2_neg_r.T, b2_neg_r.reshape(1, _D))
    return pl.pallas_call(
        _body,
        out_shape=jax.ShapeDtypeStruct((_N, 2 * _D), jnp.float32),
    )(*args)
